# Initial kernel scaffold; baseline (speedup 1.0000x reference)
import jax
import jax.numpy as jnp
from jax.experimental import pallas as pl


def kernel(x, C, Cnorm):
    raise NotImplementedError("write your pallas kernel here")


# speedup vs baseline: 1.1922x; 1.1922x over previous
import jax
import jax.numpy as jnp
from jax import lax
from jax.experimental import pallas as pl
from jax.experimental.pallas import tpu as pltpu

BM = 1024
BN = 2048


def _argmin_kernel(x_ref, c_ref, cn_ref, out_ref, best_val, best_idx):
    j = pl.program_id(1)
    nj = pl.num_programs(1)

    xb = x_ref[...]
    acc = jnp.dot(
        xb,
        c_ref[...],
        preferred_element_type=jnp.float32,
    )
    xsq = jnp.sum(xb * xb, axis=1, keepdims=True)
    scores = xsq - 2.0 * acc + cn_ref[...]

    local_min = jnp.min(scores, axis=1, keepdims=True)
    idx = lax.broadcasted_iota(jnp.int32, scores.shape, 1)
    masked = jnp.where(scores == local_min, idx, BN)
    local_arg = jnp.min(masked, axis=1, keepdims=True) + j * BN

    @pl.when(j == 0)
    def _():
        best_val[...] = local_min
        best_idx[...] = local_arg

    @pl.when(j > 0)
    def _():
        better = local_min < best_val[...]
        best_val[...] = jnp.where(better, local_min, best_val[...])
        best_idx[...] = jnp.where(better, local_arg, best_idx[...])

    @pl.when(j == nj - 1)
    def _():
        out_ref[...] = best_idx[...]


def kernel(x, C, Cnorm):
    B, T, D = x.shape
    K = C.shape[1]
    M = B * T
    x2 = x.reshape(M, D)

    grid = (M // BM, K // BN)
    out = pl.pallas_call(
        _argmin_kernel,
        grid=grid,
        in_specs=[
            pl.BlockSpec((BM, D), lambda i, j: (i, 0)),
            pl.BlockSpec((D, BN), lambda i, j: (0, j)),
            pl.BlockSpec((1, BN), lambda i, j: (0, j)),
        ],
        out_specs=pl.BlockSpec((BM, 1), lambda i, j: (i, 0)),
        out_shape=jax.ShapeDtypeStruct((M, 1), jnp.int32),
        scratch_shapes=[
            pltpu.VMEM((BM, 1), jnp.float32),
            pltpu.VMEM((BM, 1), jnp.int32),
        ],
        compiler_params=pltpu.CompilerParams(
            dimension_semantics=("parallel", "arbitrary"),
        ),
    )(x2, C, Cnorm)
    return out.reshape(B, T, 1)
